# baseline (speedup 1.0000x reference)
import dataclasses
import functools

import jax
import jax.numpy as jnp
from jax import lax
from jax.experimental import pallas as pl
from jax.experimental.pallas import tpu as pltpu
from jax.experimental.pallas import tpu_sc as plsc

NCORES = 2
NSUB = 16
LANES = 16

CHUNK = 16384
BLK = 64
BLK_SHIFT = 6
DEPTH = 4
SECS = 4


@functools.partial(jax.jit, static_argnames=("p_per_tile",))
def _scatter_add(mem, val, idx2d, p_per_tile):
    s_rows, feat = mem.shape
    nchunks = s_rows // CHUNK
    ch_per_core = nchunks // NCORES
    p = p_per_tile
    sec_p = p // SECS
    sec_nblk = (sec_p + BLK - 1) // BLK
    rows_per_tile = CHUNK // NSUB
    trash = CHUNK

    mesh = plsc.VectorSubcoreMesh(
        core_axis_name="c", subcore_axis_name="s",
        num_cores=NCORES, num_subcores=NSUB)

    cp = pltpu.CompilerParams(use_tc_tiling_on_sc=False)
    if "needs_layout_passes" in pltpu.CompilerParams.__dataclass_fields__:
        cp = dataclasses.replace(cp, needs_layout_passes=False)

    def tile_body(mem_hbm, val_hbm, idx_hbm, out_hbm, acc,
                  idx_bufs, pid_list, loc_list, rows_bufs,
                  isems, gsems, ssems):
        core = lax.axis_index("c")
        sub = lax.axis_index("s")

        ones = jnp.ones((LANES,), jnp.int32)
        zeros = jnp.zeros((LANES,), jnp.int32)
        trash_v = jnp.full((LANES,), trash, jnp.int32)
        lane = lax.iota(jnp.int32, LANES)
        last_lane = jnp.full((LANES,), LANES - 1, jnp.int32)

        my_idx = idx_hbm.at[sub]

        def start_idx_load(sec, ibuf):
            pltpu.async_copy(my_idx.at[pl.ds(sec * sec_p, sec_p)],
                             idx_bufs[ibuf], isems[ibuf])

        def wait_idx_load(ibuf):
            pltpu.make_async_copy(my_idx.at[pl.ds(0, sec_p)],
                                  idx_bufs[ibuf], isems[ibuf]).wait()

        def start_gather(b, rbuf):
            pltpu.async_copy(val_hbm.at[pid_list.at[b]],
                             rows_bufs[rbuf], gsems[rbuf])

        def wait_gather(rbuf):
            pltpu.make_async_copy(val_hbm.at[pid_list.at[0]],
                                  rows_bufs[rbuf], gsems[rbuf]).wait()

        def start_scatter(b, rbuf):
            pltpu.async_copy(rows_bufs[rbuf], acc.at[loc_list.at[b]],
                             ssems[rbuf], add=True)

        def wait_scatter(rbuf):
            pltpu.make_async_copy(rows_bufs[rbuf], acc.at[loc_list.at[0]],
                                  ssems[rbuf]).wait()

        @pl.loop(0, ch_per_core)
        def _chunk(c):
            base = (core * ch_per_core + c) * CHUNK

            start_idx_load(0, 0)
            pltpu.sync_copy(
                mem_hbm.at[pl.ds(base + sub * rows_per_tile, rows_per_tile)],
                acc.at[pl.ds(sub * rows_per_tile, rows_per_tile)])
            plsc.subcore_barrier()

            def do_section(sec):
                ib = sec % 2
                wait_idx_load(ib)
                if sec + 1 < SECS:
                    start_idx_load(sec + 1, 1 - ib)
                sec_idx = idx_bufs[ib]

                def scan_body(g, n_vec):
                    v = sec_idx[pl.ds(g * LANES, LANES)]
                    loc = v - base
                    mask = (loc >= 0) & (loc < CHUNK)
                    mi = jnp.where(mask, ones, zeros)
                    pos = n_vec + plsc.cumsum(mi) - 1
                    row = lax.shift_right_logical(pos, BLK_SHIFT)
                    col = lax.bitwise_and(pos, BLK - 1)
                    pid = sub * p + sec * sec_p + g * LANES + lane
                    plsc.store_scatter(pid_list, [row, col], pid, mask=mask)
                    plsc.store_scatter(loc_list, [row, col], loc, mask=mask)
                    return pos.at[last_lane].get(
                        mode="promise_in_bounds") + 1

                n_vec = lax.fori_loop(0, sec_p // LANES, scan_body,
                                      jnp.zeros((LANES,), jnp.int32),
                                      unroll=8)
                n = jnp.max(n_vec)

                nb = lax.shift_right_logical(n + (BLK - 1), BLK_SHIFT)
                lim = nb * BLK
                for j in range(BLK // LANES):
                    pos = n + j * LANES + lane
                    mask = pos < lim
                    row = lax.shift_right_logical(pos, BLK_SHIFT)
                    col = lax.bitwise_and(pos, BLK - 1)
                    plsc.store_scatter(pid_list, [row, col], zeros, mask=mask)
                    plsc.store_scatter(loc_list, [row, col], trash_v,
                                       mask=mask)

                for k in range(DEPTH):
                    @pl.when(k < nb)
                    def _(k=k):
                        start_gather(k, k)

                @pl.loop(0, (sec_nblk + DEPTH - 1) // DEPTH)
                def _quad(q):
                    for k in range(DEPTH):
                        b = DEPTH * q + k

                        @pl.when(b < nb)
                        def _(b=b, k=k):
                            wait_gather(k)
                            pltpu.sync_copy(rows_bufs[k],
                                            acc.at[loc_list.at[b]], add=True)

                            @pl.when(b + DEPTH < nb)
                            def _():
                                start_gather(b + DEPTH, k)

            for sec in range(SECS):
                do_section(sec)

            plsc.subcore_barrier()
            pltpu.sync_copy(
                acc.at[pl.ds(sub * rows_per_tile, rows_per_tile)],
                out_hbm.at[pl.ds(base + sub * rows_per_tile, rows_per_tile)])
            plsc.subcore_barrier()

    @pl.kernel(
        compiler_params=cp,
        out_type=jax.ShapeDtypeStruct((s_rows, feat), jnp.float32),
        mesh=mesh,
        scratch_types=[
            pltpu.VMEM_SHARED((CHUNK + 8, feat), jnp.float32),
        ],
    )
    def scatter_kernel(mem_hbm, val_hbm, idx_hbm, out_hbm, acc):
        pl.run_scoped(
            functools.partial(
                tile_body, mem_hbm, val_hbm, idx_hbm, out_hbm, acc),
            [pltpu.VMEM((sec_p,), jnp.int32) for _ in range(2)],
            pltpu.VMEM((sec_nblk, BLK), jnp.int32),
            pltpu.VMEM((sec_nblk, BLK), jnp.int32),
            [pltpu.VMEM((BLK, feat), jnp.float32) for _ in range(DEPTH)],
            [pltpu.SemaphoreType.DMA for _ in range(2)],
            [pltpu.SemaphoreType.DMA for _ in range(DEPTH)],
            [pltpu.SemaphoreType.DMA for _ in range(2)],
        )

    return scatter_kernel(mem, val, idx2d)


def kernel(mem, val, idx):
    n_pts = val.shape[0]
    groups = -(-n_pts // (NSUB * LANES * SECS))
    p_per_tile = groups * LANES * SECS
    pad = NSUB * p_per_tile - n_pts
    idx_pad = jnp.concatenate(
        [idx.astype(jnp.int32), jnp.full((pad,), 1 << 28, jnp.int32)])
    idx2d = idx_pad.reshape(NSUB, p_per_tile)
    return _scatter_add(mem, val, idx2d, p_per_tile)


# speedup vs baseline: 1.2733x; 1.2733x over previous
import dataclasses
import functools

import jax
import jax.numpy as jnp
from jax import lax
from jax.experimental import pallas as pl
from jax.experimental.pallas import tpu as pltpu
from jax.experimental.pallas import tpu_sc as plsc

NCORES = 2
NSUB = 16
LANES = 16

CHUNK = 16384
BLK = 32
BLK_SHIFT = 5
DEPTH = 8
SECS = 4


@functools.partial(jax.jit, static_argnames=("p_per_tile",))
def _scatter_add(mem, val, idx2d, p_per_tile):
    s_rows, feat = mem.shape
    nchunks = s_rows // CHUNK
    ch_per_core = nchunks // NCORES
    p = p_per_tile
    sec_p = p // SECS
    sec_nblk = (sec_p + BLK - 1) // BLK
    rows_per_tile = CHUNK // NSUB
    trash = CHUNK

    mesh = plsc.VectorSubcoreMesh(
        core_axis_name="c", subcore_axis_name="s",
        num_cores=NCORES, num_subcores=NSUB)

    cp = pltpu.CompilerParams(use_tc_tiling_on_sc=False)
    if "needs_layout_passes" in pltpu.CompilerParams.__dataclass_fields__:
        cp = dataclasses.replace(cp, needs_layout_passes=False)

    def tile_body(mem_hbm, val_hbm, idx_hbm, out_hbm, acc,
                  idx_bufs, pid_list, loc_list, rows_bufs,
                  isems, gsems, ssems):
        core = lax.axis_index("c")
        sub = lax.axis_index("s")

        ones = jnp.ones((LANES,), jnp.int32)
        zeros = jnp.zeros((LANES,), jnp.int32)
        trash_v = jnp.full((LANES,), trash, jnp.int32)
        lane = lax.iota(jnp.int32, LANES)
        last_lane = jnp.full((LANES,), LANES - 1, jnp.int32)

        my_idx = idx_hbm.at[sub]

        def start_idx_load(sec, ibuf):
            pltpu.async_copy(my_idx.at[pl.ds(sec * sec_p, sec_p)],
                             idx_bufs[ibuf], isems[ibuf])

        def wait_idx_load(ibuf):
            pltpu.make_async_copy(my_idx.at[pl.ds(0, sec_p)],
                                  idx_bufs[ibuf], isems[ibuf]).wait()

        def start_gather(b, rbuf):
            pltpu.async_copy(val_hbm.at[pid_list.at[b]],
                             rows_bufs[rbuf], gsems[rbuf])

        def wait_gather(rbuf):
            pltpu.make_async_copy(val_hbm.at[pid_list.at[0]],
                                  rows_bufs[rbuf], gsems[rbuf]).wait()

        def start_scatter(b, rbuf):
            pltpu.async_copy(rows_bufs[rbuf], acc.at[loc_list.at[b]],
                             ssems[rbuf], add=True)

        def wait_scatter(rbuf):
            pltpu.make_async_copy(rows_bufs[rbuf], acc.at[loc_list.at[0]],
                                  ssems[rbuf]).wait()

        @pl.loop(0, ch_per_core)
        def _chunk(c):
            base = (core * ch_per_core + c) * CHUNK

            start_idx_load(0, 0)
            pltpu.sync_copy(
                mem_hbm.at[pl.ds(base + sub * rows_per_tile, rows_per_tile)],
                acc.at[pl.ds(sub * rows_per_tile, rows_per_tile)])
            plsc.subcore_barrier()

            def do_section(sec):
                ib = sec % 2
                wait_idx_load(ib)
                if sec + 1 < SECS:
                    start_idx_load(sec + 1, 1 - ib)
                sec_idx = idx_bufs[ib]

                def scan_body(g, n_vec):
                    v = sec_idx[pl.ds(g * LANES, LANES)]
                    loc = v - base
                    mask = (loc >= 0) & (loc < CHUNK)
                    mi = jnp.where(mask, ones, zeros)
                    pos = n_vec + plsc.cumsum(mi) - 1
                    row = lax.shift_right_logical(pos, BLK_SHIFT)
                    col = lax.bitwise_and(pos, BLK - 1)
                    pid = sub * p + sec * sec_p + g * LANES + lane
                    plsc.store_scatter(pid_list, [row, col], pid, mask=mask)
                    plsc.store_scatter(loc_list, [row, col], loc, mask=mask)
                    return pos.at[last_lane].get(
                        mode="promise_in_bounds") + 1

                n_vec = lax.fori_loop(0, sec_p // LANES, scan_body,
                                      jnp.zeros((LANES,), jnp.int32),
                                      unroll=8)
                n = jnp.max(n_vec)

                nb = lax.shift_right_logical(n + (BLK - 1), BLK_SHIFT)
                lim = nb * BLK
                for j in range(BLK // LANES):
                    pos = n + j * LANES + lane
                    mask = pos < lim
                    row = lax.shift_right_logical(pos, BLK_SHIFT)
                    col = lax.bitwise_and(pos, BLK - 1)
                    plsc.store_scatter(pid_list, [row, col], zeros, mask=mask)
                    plsc.store_scatter(loc_list, [row, col], trash_v,
                                       mask=mask)

                for k in range(DEPTH):
                    @pl.when(k < nb)
                    def _(k=k):
                        start_gather(k, k)

                @pl.loop(0, (sec_nblk + DEPTH - 1) // DEPTH)
                def _quad(q):
                    for k in range(DEPTH):
                        b = DEPTH * q + k

                        @pl.when(b < nb)
                        def _(b=b, k=k):
                            wait_gather(k)
                            pltpu.sync_copy(rows_bufs[k],
                                            acc.at[loc_list.at[b]], add=True)

                            @pl.when(b + DEPTH < nb)
                            def _():
                                start_gather(b + DEPTH, k)

            for sec in range(SECS):
                do_section(sec)

            plsc.subcore_barrier()
            pltpu.sync_copy(
                acc.at[pl.ds(sub * rows_per_tile, rows_per_tile)],
                out_hbm.at[pl.ds(base + sub * rows_per_tile, rows_per_tile)])
            plsc.subcore_barrier()

    @pl.kernel(
        compiler_params=cp,
        out_type=jax.ShapeDtypeStruct((s_rows, feat), jnp.float32),
        mesh=mesh,
        scratch_types=[
            pltpu.VMEM_SHARED((CHUNK + 8, feat), jnp.float32),
        ],
    )
    def scatter_kernel(mem_hbm, val_hbm, idx_hbm, out_hbm, acc):
        pl.run_scoped(
            functools.partial(
                tile_body, mem_hbm, val_hbm, idx_hbm, out_hbm, acc),
            [pltpu.VMEM((sec_p,), jnp.int32) for _ in range(2)],
            pltpu.VMEM((sec_nblk, BLK), jnp.int32),
            pltpu.VMEM((sec_nblk, BLK), jnp.int32),
            [pltpu.VMEM((BLK, feat), jnp.float32) for _ in range(DEPTH)],
            [pltpu.SemaphoreType.DMA for _ in range(2)],
            [pltpu.SemaphoreType.DMA for _ in range(DEPTH)],
            [pltpu.SemaphoreType.DMA for _ in range(2)],
        )

    return scatter_kernel(mem, val, idx2d)


def kernel(mem, val, idx):
    n_pts = val.shape[0]
    groups = -(-n_pts // (NSUB * LANES * SECS))
    p_per_tile = groups * LANES * SECS
    pad = NSUB * p_per_tile - n_pts
    idx_pad = jnp.concatenate(
        [idx.astype(jnp.int32), jnp.full((pad,), 1 << 28, jnp.int32)])
    idx2d = idx_pad.reshape(NSUB, p_per_tile)
    return _scatter_add(mem, val, idx2d, p_per_tile)


# speedup vs baseline: 1.4451x; 1.1349x over previous
import dataclasses
import functools

import jax
import jax.numpy as jnp
from jax import lax
from jax.experimental import pallas as pl
from jax.experimental.pallas import tpu as pltpu
from jax.experimental.pallas import tpu_sc as plsc

NCORES = 2
NSUB = 16
LANES = 16

CHUNK = 16384
BLK = 16
BLK_SHIFT = 4
DEPTH = 16
SECS = 4


@functools.partial(jax.jit, static_argnames=("p_per_tile",))
def _scatter_add(mem, val, idx2d, p_per_tile):
    s_rows, feat = mem.shape
    nchunks = s_rows // CHUNK
    ch_per_core = nchunks // NCORES
    p = p_per_tile
    sec_p = p // SECS
    sec_nblk = (sec_p + BLK - 1) // BLK
    rows_per_tile = CHUNK // NSUB
    trash = CHUNK

    mesh = plsc.VectorSubcoreMesh(
        core_axis_name="c", subcore_axis_name="s",
        num_cores=NCORES, num_subcores=NSUB)

    cp = pltpu.CompilerParams(use_tc_tiling_on_sc=False)
    if "needs_layout_passes" in pltpu.CompilerParams.__dataclass_fields__:
        cp = dataclasses.replace(cp, needs_layout_passes=False)

    def tile_body(mem_hbm, val_hbm, idx_hbm, out_hbm, acc,
                  idx_bufs, pid_list, loc_list, rows_bufs,
                  isems, gsems, ssems):
        core = lax.axis_index("c")
        sub = lax.axis_index("s")

        ones = jnp.ones((LANES,), jnp.int32)
        zeros = jnp.zeros((LANES,), jnp.int32)
        trash_v = jnp.full((LANES,), trash, jnp.int32)
        lane = lax.iota(jnp.int32, LANES)
        last_lane = jnp.full((LANES,), LANES - 1, jnp.int32)

        my_idx = idx_hbm.at[sub]

        def start_idx_load(sec, ibuf):
            pltpu.async_copy(my_idx.at[pl.ds(sec * sec_p, sec_p)],
                             idx_bufs[ibuf], isems[ibuf])

        def wait_idx_load(ibuf):
            pltpu.make_async_copy(my_idx.at[pl.ds(0, sec_p)],
                                  idx_bufs[ibuf], isems[ibuf]).wait()

        def start_gather(b, rbuf):
            pltpu.async_copy(val_hbm.at[pid_list.at[b]],
                             rows_bufs[rbuf], gsems[rbuf])

        def wait_gather(rbuf):
            pltpu.make_async_copy(val_hbm.at[pid_list.at[0]],
                                  rows_bufs[rbuf], gsems[rbuf]).wait()

        def start_scatter(b, rbuf):
            pltpu.async_copy(rows_bufs[rbuf], acc.at[loc_list.at[b]],
                             ssems[rbuf], add=True)

        def wait_scatter(rbuf):
            pltpu.make_async_copy(rows_bufs[rbuf], acc.at[loc_list.at[0]],
                                  ssems[rbuf]).wait()

        @pl.loop(0, ch_per_core)
        def _chunk(c):
            base = (core * ch_per_core + c) * CHUNK

            start_idx_load(0, 0)
            pltpu.sync_copy(
                mem_hbm.at[pl.ds(base + sub * rows_per_tile, rows_per_tile)],
                acc.at[pl.ds(sub * rows_per_tile, rows_per_tile)])
            plsc.subcore_barrier()

            def do_section(sec):
                ib = sec % 2
                wait_idx_load(ib)
                if sec + 1 < SECS:
                    start_idx_load(sec + 1, 1 - ib)
                sec_idx = idx_bufs[ib]

                def scan_body(g, n_vec):
                    v = sec_idx[pl.ds(g * LANES, LANES)]
                    loc = v - base
                    mask = (loc >= 0) & (loc < CHUNK)
                    mi = jnp.where(mask, ones, zeros)
                    pos = n_vec + plsc.cumsum(mi) - 1
                    row = lax.shift_right_logical(pos, BLK_SHIFT)
                    col = lax.bitwise_and(pos, BLK - 1)
                    pid = sub * p + sec * sec_p + g * LANES + lane
                    plsc.store_scatter(pid_list, [row, col], pid, mask=mask)
                    plsc.store_scatter(loc_list, [row, col], loc, mask=mask)
                    return pos.at[last_lane].get(
                        mode="promise_in_bounds") + 1

                n_vec = lax.fori_loop(0, sec_p // LANES, scan_body,
                                      jnp.zeros((LANES,), jnp.int32),
                                      unroll=8)
                n = jnp.max(n_vec)

                nb = lax.shift_right_logical(n + (BLK - 1), BLK_SHIFT)
                lim = nb * BLK
                for j in range(BLK // LANES):
                    pos = n + j * LANES + lane
                    mask = pos < lim
                    row = lax.shift_right_logical(pos, BLK_SHIFT)
                    col = lax.bitwise_and(pos, BLK - 1)
                    plsc.store_scatter(pid_list, [row, col], zeros, mask=mask)
                    plsc.store_scatter(loc_list, [row, col], trash_v,
                                       mask=mask)

                for k in range(DEPTH):
                    @pl.when(k < nb)
                    def _(k=k):
                        start_gather(k, k)

                @pl.loop(0, (sec_nblk + DEPTH - 1) // DEPTH)
                def _quad(q):
                    for k in range(DEPTH):
                        b = DEPTH * q + k

                        @pl.when(b < nb)
                        def _(b=b, k=k):
                            wait_gather(k)
                            pltpu.sync_copy(rows_bufs[k],
                                            acc.at[loc_list.at[b]], add=True)

                            @pl.when(b + DEPTH < nb)
                            def _():
                                start_gather(b + DEPTH, k)

            for sec in range(SECS):
                do_section(sec)

            plsc.subcore_barrier()
            pltpu.sync_copy(
                acc.at[pl.ds(sub * rows_per_tile, rows_per_tile)],
                out_hbm.at[pl.ds(base + sub * rows_per_tile, rows_per_tile)])
            plsc.subcore_barrier()

    @pl.kernel(
        compiler_params=cp,
        out_type=jax.ShapeDtypeStruct((s_rows, feat), jnp.float32),
        mesh=mesh,
        scratch_types=[
            pltpu.VMEM_SHARED((CHUNK + 8, feat), jnp.float32),
        ],
    )
    def scatter_kernel(mem_hbm, val_hbm, idx_hbm, out_hbm, acc):
        pl.run_scoped(
            functools.partial(
                tile_body, mem_hbm, val_hbm, idx_hbm, out_hbm, acc),
            [pltpu.VMEM((sec_p,), jnp.int32) for _ in range(2)],
            pltpu.VMEM((sec_nblk, BLK), jnp.int32),
            pltpu.VMEM((sec_nblk, BLK), jnp.int32),
            [pltpu.VMEM((BLK, feat), jnp.float32) for _ in range(DEPTH)],
            [pltpu.SemaphoreType.DMA for _ in range(2)],
            [pltpu.SemaphoreType.DMA for _ in range(DEPTH)],
            [pltpu.SemaphoreType.DMA for _ in range(2)],
        )

    return scatter_kernel(mem, val, idx2d)


def kernel(mem, val, idx):
    n_pts = val.shape[0]
    groups = -(-n_pts // (NSUB * LANES * SECS))
    p_per_tile = groups * LANES * SECS
    pad = NSUB * p_per_tile - n_pts
    idx_pad = jnp.concatenate(
        [idx.astype(jnp.int32), jnp.full((pad,), 1 << 28, jnp.int32)])
    idx2d = idx_pad.reshape(NSUB, p_per_tile)
    return _scatter_add(mem, val, idx2d, p_per_tile)
